# baseline (speedup 1.0000x reference)
import functools

import jax
import jax.numpy as jnp
from jax import lax
from jax.experimental import pallas as pl
from jax.experimental.pallas import tpu as pltpu
from jax.experimental.pallas import tpu_sc as plsc

N_ITEM = 1000000
N_DIM = 64
BATCH = 16384

NC = 2
NS = 16
L = 16
NW = NC * NS
B_PER_W = BATCH // NW
CHUNK = 128
N_CHUNKS = B_PER_W // CHUNK
PAIR_W = 2 * N_DIM


def _sc_body(word_hbm, ctx_hbm, wtab_hbm, ctab_hbm, out_hbm,
             widx, cidx, wpair, cpair, wr0, wr1, cr0, cr1, out_v, sems):
    wid = lax.axis_index("s") * NC + lax.axis_index("c")
    base = wid * B_PER_W

    pltpu.sync_copy(word_hbm.at[pl.ds(base, B_PER_W)], widx)
    pltpu.sync_copy(ctx_hbm.at[pl.ds(base, B_PER_W)], cidx)

    for k in range(B_PER_W // L):
        wpair[pl.ds(k * L, L)] = widx[pl.ds(k * L, L)] >> 1
        cpair[pl.ds(k * L, L)] = cidx[pl.ds(k * L, L)] >> 1

    wbufs = [wr0, wr1]
    cbufs = [cr0, cr1]

    def fire(c):
        return (
            pltpu.async_copy(
                wtab_hbm.at[wpair.at[pl.ds(c * CHUNK, CHUNK)]],
                wbufs[c % 2], sems.at[2 * (c % 2)]),
            pltpu.async_copy(
                ctab_hbm.at[cpair.at[pl.ds(c * CHUNK, CHUNK)]],
                cbufs[c % 2], sems.at[2 * (c % 2) + 1]),
        )

    pending = {0: fire(0), 1: fire(1)}

    for c in range(N_CHUNKS):
        pending[c][0].wait()
        pending[c][1].wait()
        wbuf, cbuf = wbufs[c % 2], cbufs[c % 2]

        def body(g, _, c=c, wbuf=wbuf, cbuf=cbuf):
            gbase = c * CHUNK + g * L
            ridx = jnp.arange(L, dtype=jnp.int32) + g * L
            wcol0 = (widx[pl.ds(gbase, L)] & 1) * N_DIM
            ccol0 = (cidx[pl.ds(gbase, L)] & 1) * N_DIM
            acc = jnp.zeros((L,), jnp.float32)
            for j in range(N_DIM):
                w = plsc.load_gather(wbuf, [ridx, wcol0 + j])
                x = plsc.load_gather(cbuf, [ridx, ccol0 + j])
                acc = acc + w * x
            out_v[pl.ds(gbase, L)] = 1.0 / (1.0 + jnp.exp(-acc))
            return 0

        lax.fori_loop(0, CHUNK // L, body, 0)
        if c + 2 < N_CHUNKS:
            pending[c + 2] = fire(c + 2)

    pltpu.sync_copy(out_v, out_hbm.at[pl.ds(base, B_PER_W)])


@jax.jit
def _skipgram_sc(word, ctx, wtab2, ctab2):
    mesh = plsc.VectorSubcoreMesh(core_axis_name="c", subcore_axis_name="s",
                                  num_cores=NC, num_subcores=NS)
    return pl.kernel(
        _sc_body,
        out_type=jax.ShapeDtypeStruct((BATCH,), jnp.float32),
        mesh=mesh,
        compiler_params=pltpu.CompilerParams(needs_layout_passes=False),
        scratch_types=[
            pltpu.VMEM((B_PER_W,), jnp.int32),
            pltpu.VMEM((B_PER_W,), jnp.int32),
            pltpu.VMEM((B_PER_W,), jnp.int32),
            pltpu.VMEM((B_PER_W,), jnp.int32),
            pltpu.VMEM((CHUNK, PAIR_W), jnp.float32),
            pltpu.VMEM((CHUNK, PAIR_W), jnp.float32),
            pltpu.VMEM((CHUNK, PAIR_W), jnp.float32),
            pltpu.VMEM((CHUNK, PAIR_W), jnp.float32),
            pltpu.VMEM((B_PER_W,), jnp.float32),
            pltpu.SemaphoreType.DMA((4,)),
        ],
    )(word, ctx, wtab2, ctab2)


def kernel(word, context, word_embeddings, context_embeddings):
    wtab2 = word_embeddings.reshape(N_ITEM // 2, PAIR_W)
    ctab2 = context_embeddings.reshape(N_ITEM // 2, PAIR_W)
    return _skipgram_sc(word.astype(jnp.int32), context.astype(jnp.int32),
                        wtab2, ctab2)
